# baseline (speedup 1.0000x reference)
import functools

import jax
import jax.numpy as jnp
from jax import lax
from jax.experimental import pallas as pl
from jax.experimental.pallas import tpu as pltpu
from jax.experimental.pallas import tpu_sc as plsc

NC = 2
NS = 16
NW = NC * NS
CHUNK = 128

_mesh = plsc.VectorSubcoreMesh(core_axis_name="c", subcore_axis_name="s")


@functools.lru_cache(maxsize=None)
def _build_sc_degrees(cpw, n_pad):
    @functools.partial(
        pl.kernel,
        out_type=(jax.ShapeDtypeStruct((NW, n_pad), jnp.float32),
                  jax.ShapeDtypeStruct((NW, n_pad), jnp.float32)),
        mesh=_mesh,
        scratch_types=[
            pltpu.VMEM((cpw, CHUNK), jnp.int32),
            pltpu.VMEM((cpw, CHUNK), jnp.int32),
            pltpu.VMEM((n_pad,), jnp.float32),
            pltpu.VMEM((n_pad,), jnp.float32),
        ],
        compiler_params=pltpu.CompilerParams(needs_layout_passes=False),
    )
    def deg_kernel(src_hbm, dst_hbm, hs_out, hd_out, src_v, dst_v, hs_v, hd_v):
        cid = lax.axis_index("c")
        sid = lax.axis_index("s")
        wid = sid * NC + cid
        pltpu.sync_copy(src_hbm.at[wid], src_v)
        pltpu.sync_copy(dst_hbm.at[wid], dst_v)
        zeros16 = jnp.zeros((16,), jnp.float32)

        def zero_body(r, carry):
            hs_v[pl.ds(r * 16, 16)] = zeros16
            hd_v[pl.ds(r * 16, 16)] = zeros16
            return carry

        lax.fori_loop(0, n_pad // 16, zero_body, 0)
        ones16 = jnp.ones((16,), jnp.float32)

        def body(g, carry):
            j = g // 8
            k = g % 8
            s_idx = src_v[j, pl.ds(k * 16, 16)]
            d_idx = dst_v[j, pl.ds(k * 16, 16)]
            plsc.addupdate_scatter(hs_v, [s_idx], ones16)
            plsc.addupdate_scatter(hd_v, [d_idx], ones16)
            return carry

        lax.fori_loop(0, cpw * 8, body, 0)
        pltpu.sync_copy(hs_v, hs_out.at[wid])
        pltpu.sync_copy(hd_v, hd_out.at[wid])

    return deg_kernel


@functools.lru_cache(maxsize=None)
def _build_sc_spmm(cpw, n_pad, d):
    rows_per_tile = n_pad // NS
    nb = cpw // 8

    @functools.partial(
        pl.kernel,
        out_type=jax.ShapeDtypeStruct((NC, n_pad, d), jnp.float32),
        mesh=_mesh,
        scratch_types=[
            pltpu.VMEM((cpw, CHUNK), jnp.int32),
            pltpu.VMEM((cpw, CHUNK), jnp.int32),
            pltpu.VMEM((CHUNK, d), jnp.float32),
            pltpu.VMEM_SHARED((n_pad, d), jnp.float32),
            pltpu.SemaphoreType.DMA,
        ],
    )
    def spmm_kernel(hn_hbm, src_hbm, dst_hbm, zeros_hbm, out_hbm,
                    src_v, dst_v, rows_v, acc_sh, sem):
        cid = lax.axis_index("c")
        sid = lax.axis_index("s")
        wid = sid * NC + cid
        pltpu.sync_copy(src_hbm.at[wid], src_v)
        pltpu.sync_copy(dst_hbm.at[wid], dst_v)
        r0 = pl.multiple_of(sid * rows_per_tile, 8)
        pltpu.sync_copy(zeros_hbm.at[pl.ds(r0, rows_per_tile)],
                        acc_sh.at[pl.ds(r0, rows_per_tile)])
        plsc.subcore_barrier()

        def body(j, carry):
            pltpu.async_copy(hn_hbm.at[src_v.at[j]], rows_v, sem).wait()
            pltpu.sync_copy(rows_v, acc_sh.at[dst_v.at[j]], add=True)
            return carry

        lax.fori_loop(0, cpw, body, 0)
        plsc.subcore_barrier()
        pltpu.sync_copy(acc_sh.at[pl.ds(r0, rows_per_tile)],
                        out_hbm.at[cid, pl.ds(r0, rows_per_tile)])

    return spmm_kernel


@functools.lru_cache(maxsize=None)
def _build_tc_norms(n_pad):
    def body(hs_ref, hd_ref, ns_ref, nd_ref):
        deg_s = jnp.sum(hs_ref[...], axis=0, keepdims=True)
        deg_d = jnp.sum(hd_ref[...], axis=0, keepdims=True)
        ns_ref[...] = lax.rsqrt(jnp.maximum(deg_s, 1.0))
        nd_ref[...] = lax.rsqrt(jnp.maximum(deg_d, 1.0))

    return pl.pallas_call(
        body,
        out_shape=(jax.ShapeDtypeStruct((1, n_pad), jnp.float32),
                   jax.ShapeDtypeStruct((1, n_pad), jnp.float32)),
    )


@functools.lru_cache(maxsize=None)
def _build_tc_in(n_pad, n_real, d):
    BR = n_pad // 4
    grid = 4

    def body(x_ref, w_ref, b_ref, ns_ref, o_ref):
        i = pl.program_id(0)
        h = jnp.dot(x_ref[...], w_ref[...],
                    preferred_element_type=jnp.float32, precision=lax.Precision.HIGHEST) + b_ref[...]
        h = h * lax.logistic(h)
        h = h * ns_ref[...]
        rows = i * BR + lax.broadcasted_iota(jnp.int32, (BR, 1), 0)
        o_ref[...] = jnp.where(rows < n_real, h, 0.0)

    return pl.pallas_call(
        body,
        grid=(grid,),
        in_specs=[
            pl.BlockSpec((BR, d), lambda i: (i, 0)),
            pl.BlockSpec((d, d), lambda i: (0, 0)),
            pl.BlockSpec((1, d), lambda i: (0, 0)),
            pl.BlockSpec((BR, 1), lambda i: (i, 0)),
        ],
        out_specs=pl.BlockSpec((BR, d), lambda i: (i, 0)),
        out_shape=jax.ShapeDtypeStruct((n_pad, d), jnp.float32),
    )


@functools.lru_cache(maxsize=None)
def _build_tc_mid(n_pad, n_real, d):
    BR = n_pad // 4
    grid = 4

    def body(p_ref, nd_ref, w_ref, b_ref, ns_ref, o_ref):
        i = pl.program_id(0)
        m = (p_ref[0] + p_ref[1]) * nd_ref[...]
        h = jnp.dot(m, w_ref[...],
                    preferred_element_type=jnp.float32, precision=lax.Precision.HIGHEST) + b_ref[...]
        h = h * lax.logistic(h)
        h = h * ns_ref[...]
        rows = i * BR + lax.broadcasted_iota(jnp.int32, (BR, 1), 0)
        o_ref[...] = jnp.where(rows < n_real, h, 0.0)

    return pl.pallas_call(
        body,
        grid=(grid,),
        in_specs=[
            pl.BlockSpec((2, BR, d), lambda i: (0, i, 0)),
            pl.BlockSpec((BR, 1), lambda i: (i, 0)),
            pl.BlockSpec((d, d), lambda i: (0, 0)),
            pl.BlockSpec((1, d), lambda i: (0, 0)),
            pl.BlockSpec((BR, 1), lambda i: (i, 0)),
        ],
        out_specs=pl.BlockSpec((BR, d), lambda i: (i, 0)),
        out_shape=jax.ShapeDtypeStruct((n_pad, d), jnp.float32),
    )


@functools.lru_cache(maxsize=None)
def _build_tc_fin(n_pad, n_real, d, d_out):
    BR = n_pad // 4
    grid = 4

    def body(p_ref, nd_ref, wg_ref, bg_ref, weo_ref, beo_ref, wff_ref,
             bff_ref, o_ref, acc_ref):
        i = pl.program_id(0)
        m = (p_ref[0] + p_ref[1]) * nd_ref[...]
        h = jnp.dot(m, wg_ref[...],
                    preferred_element_type=jnp.float32, precision=lax.Precision.HIGHEST) + bg_ref[...]
        h = h * lax.logistic(h)
        rows = i * BR + lax.broadcasted_iota(jnp.int32, (BR, 1), 0)
        h = jnp.where(rows < n_real, h, 0.0)
        csum = jnp.sum(h, axis=0, keepdims=True)

        @pl.when(i == 0)
        def _():
            acc_ref[...] = csum

        @pl.when(i > 0)
        def _():
            acc_ref[...] = acc_ref[...] + csum

        @pl.when(i == grid - 1)
        def _():
            pooled = jnp.dot(acc_ref[...], weo_ref[...],
                             preferred_element_type=jnp.float32, precision=lax.Precision.HIGHEST)
            pooled = pooled + jnp.float32(n_real) * beo_ref[...]
            o_ref[...] = jnp.dot(pooled, wff_ref[...],
                                 preferred_element_type=jnp.float32, precision=lax.Precision.HIGHEST) + bff_ref[...]

    return pl.pallas_call(
        body,
        grid=(grid,),
        in_specs=[
            pl.BlockSpec((2, BR, d), lambda i: (0, i, 0)),
            pl.BlockSpec((BR, 1), lambda i: (i, 0)),
            pl.BlockSpec((d, d), lambda i: (0, 0)),
            pl.BlockSpec((1, d), lambda i: (0, 0)),
            pl.BlockSpec((d, d), lambda i: (0, 0)),
            pl.BlockSpec((1, d), lambda i: (0, 0)),
            pl.BlockSpec((d, d_out), lambda i: (0, 0)),
            pl.BlockSpec((1, d_out), lambda i: (0, 0)),
        ],
        out_specs=pl.BlockSpec((1, d_out), lambda i: (0, 0)),
        out_shape=jax.ShapeDtypeStruct((1, d_out), jnp.float32),
        scratch_shapes=[pltpu.VMEM((1, d), jnp.float32)],
    )


def kernel(x, edge_index, W_in, b_in, W_g0, b_g0, W_g1, b_g1, W_g2, b_g2,
           W_eo, b_eo, W_ff, b_ff):
    n_real, d = x.shape
    d_out = W_ff.shape[1]
    e = edge_index.shape[1]
    n_pad = -(-(n_real + 1) // 128) * 128
    e_pad = -(-e // (16 * NW * CHUNK)) * (16 * NW * CHUNK)
    cpw = e_pad // (NW * CHUNK)

    src = edge_index[0].astype(jnp.int32)
    dst = edge_index[1].astype(jnp.int32)
    pad_idx = jnp.full((e_pad - e,), n_real, jnp.int32)
    src_r = jnp.concatenate([src, pad_idx]).reshape(NW, cpw, CHUNK)
    dst_r = jnp.concatenate([dst, pad_idx]).reshape(NW, cpw, CHUNK)
    zeros = jnp.zeros((n_pad, d), jnp.float32)
    x_pad = jnp.concatenate(
        [x.astype(jnp.float32), jnp.zeros((n_pad - n_real, d), jnp.float32)], 0)

    hs, hd = _build_sc_degrees(cpw, n_pad)(src_r, dst_r)
    ns2, nd2 = _build_tc_norms(n_pad)(hs, hd)
    norm_src = ns2.reshape(n_pad, 1)
    norm_dst = nd2.reshape(n_pad, 1)

    hn = _build_tc_in(n_pad, n_real, d)(x_pad, W_in, b_in.reshape(1, d),
                                        norm_src)
    spmm = _build_sc_spmm(cpw, n_pad, d)
    mid = _build_tc_mid(n_pad, n_real, d)
    for (W, b) in ((W_g0, b_g0), (W_g1, b_g1)):
        parts = spmm(hn, src_r, dst_r, zeros)
        hn = mid(parts, norm_dst, W, b.reshape(1, d), norm_src)
    parts = spmm(hn, src_r, dst_r, zeros)
    out = _build_tc_fin(n_pad, n_real, d, d_out)(
        parts, norm_dst, W_g2, b_g2.reshape(1, d), W_eo, b_eo.reshape(1, d),
        W_ff.reshape(d, d_out), b_ff.reshape(1, d_out))
    return out


# speedup vs baseline: 1.7814x; 1.7814x over previous
import functools

import jax
import jax.numpy as jnp
from jax import lax
from jax.experimental import pallas as pl
from jax.experimental.pallas import tpu as pltpu
from jax.experimental.pallas import tpu_sc as plsc

NC = 2
NS = 16
NW = NC * NS
CHUNK = 128

_mesh = plsc.VectorSubcoreMesh(core_axis_name="c", subcore_axis_name="s")


@functools.lru_cache(maxsize=None)
def _build_sc_degrees(cpw, n_pad):
    @functools.partial(
        pl.kernel,
        out_type=(jax.ShapeDtypeStruct((NW, n_pad), jnp.float32),
                  jax.ShapeDtypeStruct((NW, n_pad), jnp.float32)),
        mesh=_mesh,
        scratch_types=[
            pltpu.VMEM((cpw, CHUNK), jnp.int32),
            pltpu.VMEM((cpw, CHUNK), jnp.int32),
            pltpu.VMEM((n_pad,), jnp.float32),
            pltpu.VMEM((n_pad,), jnp.float32),
        ],
        compiler_params=pltpu.CompilerParams(needs_layout_passes=False),
    )
    def deg_kernel(src_hbm, dst_hbm, hs_out, hd_out, src_v, dst_v, hs_v, hd_v):
        cid = lax.axis_index("c")
        sid = lax.axis_index("s")
        wid = sid * NC + cid
        pltpu.sync_copy(src_hbm.at[wid], src_v)
        pltpu.sync_copy(dst_hbm.at[wid], dst_v)
        zeros16 = jnp.zeros((16,), jnp.float32)

        def zero_body(r, carry):
            hs_v[pl.ds(r * 16, 16)] = zeros16
            hd_v[pl.ds(r * 16, 16)] = zeros16
            return carry

        lax.fori_loop(0, n_pad // 16, zero_body, 0)
        ones16 = jnp.ones((16,), jnp.float32)

        def body(g, carry):
            j = g // 8
            k = g % 8
            s_idx = src_v[j, pl.ds(k * 16, 16)]
            d_idx = dst_v[j, pl.ds(k * 16, 16)]
            plsc.addupdate_scatter(hs_v, [s_idx], ones16)
            plsc.addupdate_scatter(hd_v, [d_idx], ones16)
            return carry

        lax.fori_loop(0, cpw * 8, body, 0)
        pltpu.sync_copy(hs_v, hs_out.at[wid])
        pltpu.sync_copy(hd_v, hd_out.at[wid])

    return deg_kernel


@functools.lru_cache(maxsize=None)
def _build_sc_spmm(cpw, n_pad, d):
    rows_per_tile = n_pad // NS
    nb = cpw // 8

    @functools.partial(
        pl.kernel,
        out_type=jax.ShapeDtypeStruct((NC, n_pad, d), jnp.float32),
        mesh=_mesh,
        scratch_types=[
            pltpu.VMEM((cpw, CHUNK), jnp.int32),
            pltpu.VMEM((cpw, CHUNK), jnp.int32),
            pltpu.VMEM((CHUNK, d), jnp.float32),
            pltpu.VMEM_SHARED((n_pad, d), jnp.float32),
            pltpu.SemaphoreType.DMA,
        ],
    )
    def spmm_kernel(hn_hbm, src_hbm, dst_hbm, zeros_hbm, out_hbm,
                    src_v, dst_v, rows_v, acc_sh, sem):
        cid = lax.axis_index("c")
        sid = lax.axis_index("s")
        wid = sid * NC + cid
        pltpu.sync_copy(src_hbm.at[wid], src_v)
        pltpu.sync_copy(dst_hbm.at[wid], dst_v)
        r0 = pl.multiple_of(sid * rows_per_tile, 8)
        pltpu.sync_copy(zeros_hbm.at[pl.ds(r0, rows_per_tile)],
                        acc_sh.at[pl.ds(r0, rows_per_tile)])
        plsc.subcore_barrier()

        def body(j, carry):
            pltpu.async_copy(hn_hbm.at[src_v.at[j]], rows_v, sem).wait()
            pltpu.sync_copy(rows_v, acc_sh.at[dst_v.at[j]], add=True)
            return carry

        lax.fori_loop(0, cpw, body, 0)
        plsc.subcore_barrier()
        pltpu.sync_copy(acc_sh.at[pl.ds(r0, rows_per_tile)],
                        out_hbm.at[cid, pl.ds(r0, rows_per_tile)])

    return spmm_kernel


@functools.lru_cache(maxsize=None)
def _build_tc_norms(n_pad):
    def body(hs_ref, hd_ref, ns_ref, nd_ref):
        deg_s = jnp.sum(hs_ref[...], axis=0, keepdims=True)
        deg_d = jnp.sum(hd_ref[...], axis=0, keepdims=True)
        ns_ref[...] = lax.rsqrt(jnp.maximum(deg_s, 1.0))
        nd_ref[...] = lax.rsqrt(jnp.maximum(deg_d, 1.0))

    return pl.pallas_call(
        body,
        out_shape=(jax.ShapeDtypeStruct((1, n_pad), jnp.float32),
                   jax.ShapeDtypeStruct((1, n_pad), jnp.float32)),
    )


@functools.lru_cache(maxsize=None)
def _build_tc_in(n_pad, n_real, d):
    BR = 1024
    grid = n_pad // BR

    def body(x_ref, w_ref, b_ref, ns_ref, o_ref):
        i = pl.program_id(0)
        h = jnp.dot(x_ref[...], w_ref[...],
                    preferred_element_type=jnp.float32, precision=lax.Precision.HIGHEST) + b_ref[...]
        h = h * lax.logistic(h)
        h = h * ns_ref[...]
        rows = i * BR + lax.broadcasted_iota(jnp.int32, (BR, 1), 0)
        o_ref[...] = jnp.where(rows < n_real, h, 0.0)

    return pl.pallas_call(
        body,
        grid=(grid,),
        in_specs=[
            pl.BlockSpec((BR, d), lambda i: (i, 0)),
            pl.BlockSpec((d, d), lambda i: (0, 0)),
            pl.BlockSpec((1, d), lambda i: (0, 0)),
            pl.BlockSpec((BR, 1), lambda i: (i, 0)),
        ],
        out_specs=pl.BlockSpec((BR, d), lambda i: (i, 0)),
        out_shape=jax.ShapeDtypeStruct((n_pad, d), jnp.float32),
    )


@functools.lru_cache(maxsize=None)
def _build_tc_mid(n_pad, n_real, d):
    BR = 1024
    grid = n_pad // BR

    def body(p_ref, nd_ref, w_ref, b_ref, ns_ref, o_ref):
        i = pl.program_id(0)
        m = (p_ref[0] + p_ref[1]) * nd_ref[...]
        h = jnp.dot(m, w_ref[...],
                    preferred_element_type=jnp.float32, precision=lax.Precision.HIGHEST) + b_ref[...]
        h = h * lax.logistic(h)
        h = h * ns_ref[...]
        rows = i * BR + lax.broadcasted_iota(jnp.int32, (BR, 1), 0)
        o_ref[...] = jnp.where(rows < n_real, h, 0.0)

    return pl.pallas_call(
        body,
        grid=(grid,),
        in_specs=[
            pl.BlockSpec((2, BR, d), lambda i: (0, i, 0)),
            pl.BlockSpec((BR, 1), lambda i: (i, 0)),
            pl.BlockSpec((d, d), lambda i: (0, 0)),
            pl.BlockSpec((1, d), lambda i: (0, 0)),
            pl.BlockSpec((BR, 1), lambda i: (i, 0)),
        ],
        out_specs=pl.BlockSpec((BR, d), lambda i: (i, 0)),
        out_shape=jax.ShapeDtypeStruct((n_pad, d), jnp.float32),
    )


@functools.lru_cache(maxsize=None)
def _build_tc_fin(n_pad, n_real, d, d_out):
    BR = 1024
    grid = n_pad // BR

    def body(p_ref, nd_ref, wg_ref, bg_ref, weo_ref, beo_ref, wff_ref,
             bff_ref, o_ref, acc_ref):
        i = pl.program_id(0)
        m = (p_ref[0] + p_ref[1]) * nd_ref[...]
        h = jnp.dot(m, wg_ref[...],
                    preferred_element_type=jnp.float32, precision=lax.Precision.HIGHEST) + bg_ref[...]
        h = h * lax.logistic(h)
        rows = i * BR + lax.broadcasted_iota(jnp.int32, (BR, 1), 0)
        h = jnp.where(rows < n_real, h, 0.0)
        csum = jnp.sum(h, axis=0, keepdims=True)

        @pl.when(i == 0)
        def _():
            acc_ref[...] = csum

        @pl.when(i > 0)
        def _():
            acc_ref[...] = acc_ref[...] + csum

        @pl.when(i == grid - 1)
        def _():
            pooled = jnp.dot(acc_ref[...], weo_ref[...],
                             preferred_element_type=jnp.float32, precision=lax.Precision.HIGHEST)
            pooled = pooled + jnp.float32(n_real) * beo_ref[...]
            o_ref[...] = jnp.dot(pooled, wff_ref[...],
                                 preferred_element_type=jnp.float32, precision=lax.Precision.HIGHEST) + bff_ref[...]

    return pl.pallas_call(
        body,
        grid=(grid,),
        in_specs=[
            pl.BlockSpec((2, BR, d), lambda i: (0, i, 0)),
            pl.BlockSpec((BR, 1), lambda i: (i, 0)),
            pl.BlockSpec((d, d), lambda i: (0, 0)),
            pl.BlockSpec((1, d), lambda i: (0, 0)),
            pl.BlockSpec((d, d), lambda i: (0, 0)),
            pl.BlockSpec((1, d), lambda i: (0, 0)),
            pl.BlockSpec((d, d_out), lambda i: (0, 0)),
            pl.BlockSpec((1, d_out), lambda i: (0, 0)),
        ],
        out_specs=pl.BlockSpec((1, d_out), lambda i: (0, 0)),
        out_shape=jax.ShapeDtypeStruct((1, d_out), jnp.float32),
        scratch_shapes=[pltpu.VMEM((1, d), jnp.float32)],
    )


def kernel(x, edge_index, W_in, b_in, W_g0, b_g0, W_g1, b_g1, W_g2, b_g2,
           W_eo, b_eo, W_ff, b_ff):
    n_real, d = x.shape
    d_out = W_ff.shape[1]
    e = edge_index.shape[1]
    n_pad = -(-(n_real + 1) // 1024) * 1024
    e_pad = -(-e // (NW * CHUNK)) * (NW * CHUNK)
    cpw = e_pad // (NW * CHUNK)

    src = edge_index[0].astype(jnp.int32)
    dst = edge_index[1].astype(jnp.int32)
    pad_idx = jnp.full((e_pad - e,), n_real, jnp.int32)
    src_r = jnp.concatenate([src, pad_idx]).reshape(NW, cpw, CHUNK)
    dst_r = jnp.concatenate([dst, pad_idx]).reshape(NW, cpw, CHUNK)
    zeros = jnp.zeros((n_pad, d), jnp.float32)
    x_pad = jnp.concatenate(
        [x.astype(jnp.float32), jnp.zeros((n_pad - n_real, d), jnp.float32)], 0)

    hs, hd = _build_sc_degrees(cpw, n_pad)(src_r, dst_r)
    ns2, nd2 = _build_tc_norms(n_pad)(hs, hd)
    norm_src = ns2.reshape(n_pad, 1)
    norm_dst = nd2.reshape(n_pad, 1)

    hn = _build_tc_in(n_pad, n_real, d)(x_pad, W_in, b_in.reshape(1, d),
                                        norm_src)
    spmm = _build_sc_spmm(cpw, n_pad, d)
    mid = _build_tc_mid(n_pad, n_real, d)
    for (W, b) in ((W_g0, b_g0), (W_g1, b_g1)):
        parts = spmm(hn, src_r, dst_r, zeros)
        hn = mid(parts, norm_dst, W, b.reshape(1, d), norm_src)
    parts = spmm(hn, src_r, dst_r, zeros)
    out = _build_tc_fin(n_pad, n_real, d, d_out)(
        parts, norm_dst, W_g2, b_g2.reshape(1, d), W_eo, b_eo.reshape(1, d),
        W_ff.reshape(d, d_out), b_ff.reshape(1, d_out))
    return out
